# baseline (speedup 1.0000x reference)
import jax
import jax.numpy as jnp
from jax.experimental import pallas as pl
from jax.experimental.pallas import tpu as pltpu

_B = 1024
_NB_NODES = 32767
_N_PAD = 32768
_TILE = 8192
_B_BLK = 256


def _tile_kernel(x_ref, a1m_ref, a1x_ref, a1p_ref, minf_ref, o_ref, wbig_ref, wpar_ref):
    t = pl.program_id(0)
    b = pl.program_id(1)

    @pl.when(b == 0)
    def _build_weights():
        am = a1m_ref[...]
        ax = a1x_ref[...]
        shifted = jnp.concatenate([am[1:], ax[:1]], axis=0)
        wbig_ref[...] = jnp.stack([-am, shifted], axis=1).reshape(_TILE, am.shape[1])
        wpar_ref[...] = jnp.repeat(a1p_ref[...], 4, axis=0)

    x = x_ref[...]
    w = jnp.dot(x, wbig_ref[...].T,
                preferred_element_type=jnp.float32)
    p = jnp.dot(x, wpar_ref[...].T,
                preferred_element_type=jnp.float32)
    z = jnp.clip(jnp.minimum(p + minf_ref[...], w), 0.0, 1.0)
    o_ref[...] = z

    @pl.when(t == 0)
    def _root():
        o_ref[:, 0:1] = jnp.ones((x_ref.shape[0], 1), jnp.float32)


def kernel(X, A, split_nodes, desc_left, desc_right):
    del split_nodes, desc_left, desc_right
    dim = A.shape[1]
    a1 = jnp.concatenate([jnp.zeros((1, dim), A.dtype), A], axis=0)
    minf = jnp.where(jnp.arange(_N_PAD) % 4 == 0, 0.0, jnp.inf)
    minf = minf.astype(jnp.float32)[None, :]

    n_tiles = _N_PAD // _TILE
    n_b = _B // _B_BLK
    ext_blk = (_TILE // 2) // 8
    out = pl.pallas_call(
        _tile_kernel,
        grid=(n_tiles, n_b),
        in_specs=[
            pl.BlockSpec((_B_BLK, dim), lambda t, b: (b, 0)),
            pl.BlockSpec((_TILE // 2, dim), lambda t, b: (t, 0)),
            pl.BlockSpec((8, dim), lambda t, b: (ext_blk * (t + 1), 0)),
            pl.BlockSpec((_TILE // 4, dim), lambda t, b: (t, 0)),
            pl.BlockSpec((1, _TILE), lambda t, b: (0, t)),
        ],
        out_specs=pl.BlockSpec((_B_BLK, _TILE), lambda t, b: (b, t)),
        out_shape=jax.ShapeDtypeStruct((_B, _NB_NODES), jnp.float32),
        scratch_shapes=[
            pltpu.VMEM((_TILE, dim), jnp.float32),
            pltpu.VMEM((_TILE, dim), jnp.float32),
        ],
    )(X, a1, a1, a1, minf)
    return out


# speedup vs baseline: 1.0154x; 1.0154x over previous
import jax
import jax.numpy as jnp
from jax.experimental import pallas as pl
from jax.experimental.pallas import tpu as pltpu

_B = 1024
_NB_NODES = 32767
_N_PAD = 32768
_TILE = 4096
_B_BLK = 512


def _tile_kernel(x_ref, a1m_ref, a1x_ref, a1p_ref, o_ref, wbig_ref, wpar_ref):
    t = pl.program_id(0)
    b = pl.program_id(1)

    @pl.when(b == 0)
    def _build_weights():
        am = a1m_ref[...]
        ax = a1x_ref[...]
        shifted = jnp.concatenate([am[1:], ax[:1]], axis=0)
        wbig_ref[...] = jnp.stack([-am, shifted], axis=1).reshape(_TILE, am.shape[1])
        wpar_ref[...] = jnp.repeat(a1p_ref[...], 4, axis=0)

    x = x_ref[...]
    w = jnp.dot(x, wbig_ref[...].T,
                preferred_element_type=jnp.float32)
    p = jnp.dot(x, wpar_ref[...].T,
                preferred_element_type=jnp.float32)
    j_loc = jax.lax.broadcasted_iota(jnp.int32, w.shape, 1)
    z = jnp.where(j_loc % 4 == 0, jnp.minimum(p, w), w)
    z = jnp.clip(z, 0.0, 1.0)
    z = jnp.where(jnp.logical_and(t == 0, j_loc == 0), 1.0, z)
    o_ref[...] = z


def kernel(X, A, split_nodes, desc_left, desc_right):
    del split_nodes, desc_left, desc_right
    dim = A.shape[1]
    a1 = jnp.concatenate([jnp.zeros((1, dim), A.dtype), A], axis=0)

    n_tiles = _N_PAD // _TILE
    n_b = _B // _B_BLK
    ext_blk = (_TILE // 2) // 8
    out = pl.pallas_call(
        _tile_kernel,
        grid=(n_tiles, n_b),
        in_specs=[
            pl.BlockSpec((_B_BLK, dim), lambda t, b: (b, 0)),
            pl.BlockSpec((_TILE // 2, dim), lambda t, b: (t, 0)),
            pl.BlockSpec((8, dim), lambda t, b: (ext_blk * (t + 1), 0)),
            pl.BlockSpec((_TILE // 4, dim), lambda t, b: (t, 0)),
        ],
        out_specs=pl.BlockSpec((_B_BLK, _TILE), lambda t, b: (b, t)),
        out_shape=jax.ShapeDtypeStruct((_B, _NB_NODES), jnp.float32),
        scratch_shapes=[
            pltpu.VMEM((_TILE, dim), jnp.float32),
            pltpu.VMEM((_TILE, dim), jnp.float32),
        ],
    )(X, a1, a1, a1)
    return out
